# baseline (speedup 1.0000x reference)
import functools

import jax
import jax.numpy as jnp
from jax import lax
from jax.experimental import pallas as pl
from jax.experimental.pallas import tpu as pltpu
from jax.experimental.pallas import tpu_sc as plsc

NC, NS, LANES = 2, 16, 16
NW = NC * NS
GROUPS = 8
EPS = 1e-5
NTYPES = 7


def _gn(h, gns, gnb, C):
    g = C // GROUPS
    r = lax.broadcasted_iota(jnp.int32, (C, C), 0) // g
    c = lax.broadcasted_iota(jnp.int32, (C, C), 1) // g
    mg = (r == c).astype(jnp.float32) / g
    m = jnp.dot(h, mg, preferred_element_type=jnp.float32)
    e2 = jnp.dot(h * h, mg, preferred_element_type=jnp.float32)
    v = e2 - m * m
    return (h - m) * lax.rsqrt(v + EPS) * gns + gnb


def _down(xin, w_ref, b_ref, gns_ref, gnb_ref, C):
    n8, cin = xin.shape
    xp = jnp.mean(xin.reshape(n8 // 8, 8, cin), axis=1)
    h = jnp.dot(xp, w_ref[:], preferred_element_type=jnp.float32) + b_ref[:]
    return jax.nn.gelu(_gn(h, gns_ref[:], gnb_ref[:], C))


def _table(x, nt_ref, wx_ref, wo_ref, bcat_ref, NT):
    oh = (nt_ref[:] == lax.broadcasted_iota(jnp.int32, (1, NT), 1))
    oh = oh.astype(jnp.float32)
    return (jnp.dot(x, wx_ref[:], preferred_element_type=jnp.float32)
            + jnp.dot(oh, wo_ref[:], preferred_element_type=jnp.float32)
            + bcat_ref[:])


def _tc_stage0_body(nt_ref, d_ref, w_ref, b_ref, gns_ref, gnb_ref,
                    wx_ref, wo_ref, bcat_ref, out_ref, *, C, NT):
    h = jnp.dot(d_ref[:], w_ref[:], preferred_element_type=jnp.float32)
    h = h + b_ref[:]
    x = jax.nn.gelu(_gn(h, gns_ref[:], gnb_ref[:], C))
    out_ref[:] = _table(x, nt_ref, wx_ref, wo_ref, bcat_ref, NT)


def _tc_stage0(nt2, pooled, w, b, gns, gnb, wx, wo, bcat, *, C, NT, bn):
    N, CIN = pooled.shape
    TW = wx.shape[1]
    return pl.pallas_call(
        functools.partial(_tc_stage0_body, C=C, NT=NT),
        grid=(N // bn,),
        in_specs=[
            pl.BlockSpec((bn, 1), lambda i: (i, 0)),
            pl.BlockSpec((bn, CIN), lambda i: (i, 0)),
            pl.BlockSpec((CIN, C), lambda i: (0, 0)),
            pl.BlockSpec((1, C), lambda i: (0, 0)),
            pl.BlockSpec((1, C), lambda i: (0, 0)),
            pl.BlockSpec((1, C), lambda i: (0, 0)),
            pl.BlockSpec((C, TW), lambda i: (0, 0)),
            pl.BlockSpec((NT, TW), lambda i: (0, 0)),
            pl.BlockSpec((1, TW), lambda i: (0, 0)),
        ],
        out_specs=pl.BlockSpec((bn, TW), lambda i: (i, 0)),
        out_shape=jax.ShapeDtypeStruct((N, TW), jnp.float32),
    )(nt2, pooled, w, b, gns, gnb, wx, wo, bcat)


def _tc_mid_body(nt_ref, p0_ref, p1_ref, gnsc_ref, gnbc_ref,
                 w_ref, b_ref, gns_ref, gnb_ref, wx_ref, wo_ref, bcat_ref,
                 out_ref, *, C0, C1, NT):
    s = p0_ref[:] + p1_ref[:]
    deg = jnp.maximum(s[:, C0:C0 + 1], 1.0)
    agg = s[:, :C0] / deg
    x1 = jax.nn.gelu(_gn(agg, gnsc_ref[:], gnbc_ref[:], C0))
    x2 = _down(x1, w_ref, b_ref, gns_ref, gnb_ref, C1)
    out_ref[:] = _table(x2, nt_ref, wx_ref, wo_ref, bcat_ref, NT)


def _tc_mid(nt2, p0, p1, gnsc, gnbc, w, b, gns, gnb, wx, wo, bcat,
            *, C0, C1, NT):
    N1, W = p0.shape
    N2 = N1 // 8
    TW = wx.shape[1]
    return pl.pallas_call(
        functools.partial(_tc_mid_body, C0=C0, C1=C1, NT=NT),
        out_shape=jax.ShapeDtypeStruct((N2, TW), jnp.float32),
    )(nt2, p0, p1, gnsc, gnbc, w, b, gns, gnb, wx, wo, bcat)


def _tc_final_body(p0_ref, p1_ref, gns_ref, gnb_ref, out_ref, *, C):
    s = p0_ref[:] + p1_ref[:]
    deg = jnp.maximum(s[:, C:C + 1], 1.0)
    agg = s[:, :C] / deg
    out_ref[:] = jax.nn.gelu(_gn(agg, gns_ref[:], gnb_ref[:], C))


def _tc_final(p0, p1, gns, gnb, *, C):
    N, W = p0.shape
    return pl.pallas_call(
        functools.partial(_tc_final_body, C=C),
        out_shape=jax.ShapeDtypeStruct((N, C), jnp.float32),
    )(p0, p1, gns, gnb)


def _sc_conv(xt2d, ei3, et2, *, NACC, WIDTH, B=128, NBUF=8):
    NTAB, TW = xt2d.shape
    E = ei3.shape[1] * B
    ew = E // NW
    nchunk = ew // B
    nouter = nchunk // NBUF
    rps = NACC // NS
    mesh = plsc.VectorSubcoreMesh(core_axis_name="c", subcore_axis_name="s",
                                  num_cores=NC, num_subcores=NS)

    @functools.partial(
        pl.kernel,
        out_type=jax.ShapeDtypeStruct((NC, NACC, WIDTH), jnp.float32),
        mesh=mesh,
        compiler_params=pltpu.CompilerParams(use_tc_tiling_on_sc=False),
        scratch_types=[
            pltpu.VMEM((nchunk, B), jnp.int32),
            pltpu.VMEM((nchunk, B), jnp.int32),
            pltpu.VMEM((nchunk, B), jnp.int32),
            pltpu.VMEM((nchunk, B), jnp.int32),
            pltpu.VMEM((NBUF, B, WIDTH), jnp.float32),
            pltpu.VMEM_SHARED((NACC, WIDTH), jnp.float32),
            [pltpu.SemaphoreType.DMA] * NBUF,
            [pltpu.SemaphoreType.DMA] * NBUF,
        ],
    )
    def k(xt_hbm, ei_hbm, et_hbm, out_hbm,
          srcb, typb, dstb, idxb, rowsb, acc, gsem, ssem):
        cid = lax.axis_index("c")
        sid = lax.axis_index("s")
        wid = sid * NC + cid
        row0 = wid * nchunk
        xtr = xt_hbm

        pltpu.sync_copy(ei_hbm.at[0, pl.ds(row0, nchunk)], srcb)
        pltpu.sync_copy(ei_hbm.at[1, pl.ds(row0, nchunk)], dstb)
        pltpu.sync_copy(et_hbm.at[pl.ds(row0, nchunk)], typb)

        @pl.loop(0, B)
        def _(i):
            for cc in range(WIDTH // LANES):
                rowsb[0, i, pl.ds(cc * LANES, LANES)] = jnp.zeros(
                    (LANES,), jnp.float32)
        rbase = sid * rps
        for kk in range(rps // B):
            pltpu.sync_copy(rowsb.at[0], acc.at[pl.ds(rbase + kk * B, B)])

        @pl.loop(0, nchunk)
        def _(c):
            for j in range(B // LANES):
                sl = pl.ds(j * LANES, LANES)
                idxb[c, sl] = srcb[c, sl] * NTYPES + typb[c, sl]
        plsc.subcore_barrier()

        def fire_gather(c, b):
            pltpu.async_copy(xtr.at[idxb.at[c]], rowsb.at[b], gsem[b])

        def wait_gather(b):
            pltpu.make_async_copy(
                xtr.at[idxb.at[0]], rowsb.at[b], gsem[b]).wait()

        def fire_scatter(c, b):
            pltpu.async_copy(rowsb.at[b], acc.at[dstb.at[c]], ssem[b],
                             add=True)

        def wait_scatter(b):
            pltpu.make_async_copy(
                rowsb.at[b], acc.at[dstb.at[0]], ssem[b]).wait()

        for b in range(NBUF):
            fire_gather(b, b)

        @pl.loop(0, nouter - 1)
        def _(i):
            c0 = i * NBUF
            for b in range(NBUF):
                wait_gather(b)
                fire_scatter(c0 + b, b)
            for b in range(NBUF):
                wait_scatter(b)
                fire_gather(c0 + NBUF + b, b)

        c0 = (nouter - 1) * NBUF
        for b in range(NBUF):
            wait_gather(b)
            fire_scatter(c0 + b, b)
        for b in range(NBUF):
            wait_scatter(b)

        plsc.subcore_barrier()
        for kk in range(rps // B):
            r0 = rbase + kk * B
            pltpu.sync_copy(acc.at[pl.ds(r0, B)], rowsb.at[0])
            pltpu.sync_copy(rowsb.at[0], out_hbm.at[cid, pl.ds(r0, B)])

    return k(xt2d, ei3, et2)


def _sc_pool(data, *, NOUT, NBUF=8, CH=128):
    N0, C = data.shape
    rpw = N0 // NW
    opw = rpw // 8
    nchunk = rpw // CH
    opc = CH // 8
    nouter = nchunk // NBUF
    mesh = plsc.VectorSubcoreMesh(core_axis_name="c", subcore_axis_name="s",
                                  num_cores=NC, num_subcores=NS)

    @functools.partial(
        pl.kernel,
        out_type=jax.ShapeDtypeStruct((NOUT, C), jnp.float32),
        mesh=mesh,
        compiler_params=pltpu.CompilerParams(use_tc_tiling_on_sc=False),
        scratch_types=[
            pltpu.VMEM((nchunk, CH), jnp.int32),
            pltpu.VMEM((NBUF, CH, C), jnp.float32),
            pltpu.VMEM((opw, C), jnp.float32),
            pltpu.VMEM_SHARED((NOUT // NC, C), jnp.float32),
            [pltpu.SemaphoreType.DMA] * NBUF,
            [pltpu.SemaphoreType.DMA] * NBUF,
        ],
    )
    def k(d_hbm, out_hbm, idxb, ring, stage, acc, lsem, ssem):
        cid = lax.axis_index("c")
        sid = lax.axis_index("s")
        in_base = cid * (N0 // NC) + sid * rpw
        obase = sid * opw

        lane8 = lax.shift_right_logical(lax.iota(jnp.int32, LANES), 3)

        @pl.loop(0, nchunk)
        def _(c):
            for kk in range(CH // LANES):
                idxb[c, pl.ds(kk * LANES, LANES)] = (
                    lane8 + (obase + c * opc + 2 * kk))

        @pl.loop(0, opw)
        def _(i):
            for cc in range(C // LANES):
                stage[i, pl.ds(cc * LANES, LANES)] = jnp.zeros(
                    (LANES,), jnp.float32)
        pltpu.sync_copy(stage, acc.at[pl.ds(obase, opw)])

        def fire_load(c, b):
            pltpu.async_copy(d_hbm.at[pl.ds(in_base + c * CH, CH)],
                             ring.at[b], lsem[b])

        def wait_load(b):
            pltpu.make_async_copy(d_hbm.at[pl.ds(0, CH)], ring.at[b],
                                  lsem[b]).wait()

        def fire_add(c, b):
            pltpu.async_copy(ring.at[b], acc.at[idxb.at[c]], ssem[b],
                             add=True)

        def wait_add(b):
            pltpu.make_async_copy(ring.at[b], acc.at[idxb.at[0]],
                                  ssem[b]).wait()

        for b in range(NBUF):
            fire_load(b, b)

        @pl.loop(0, nouter - 1)
        def _(i):
            c0 = i * NBUF
            for b in range(NBUF):
                wait_load(b)
                fire_add(c0 + b, b)
            for b in range(NBUF):
                wait_add(b)
                fire_load(c0 + NBUF + b, b)

        c0 = (nouter - 1) * NBUF
        for b in range(NBUF):
            wait_load(b)
            fire_add(c0 + b, b)
        for b in range(NBUF):
            wait_add(b)

        pltpu.sync_copy(acc.at[pl.ds(obase, opw)], stage)
        pltpu.sync_copy(stage, out_hbm.at[pl.ds(cid * (NOUT // NC) + obase,
                                                opw)])

    return k(data)


def _prep_conv_weights(W_conv, C, NT, WPAD):
    Cout = W_conv.shape[2]
    wx = jnp.transpose(W_conv[:, :C, :], (1, 0, 2))
    wx = jnp.pad(wx, ((0, 0), (0, 0), (0, WPAD - Cout)))
    wx = wx.reshape(C, NTYPES * WPAD)
    wo = jnp.transpose(W_conv[:, C:, :], (1, 0, 2))
    wo = jnp.pad(wo, ((0, 0), (0, 0), (0, WPAD - Cout)))
    wo = wo.reshape(NT, NTYPES * WPAD)
    bcat = jnp.zeros((NTYPES, WPAD), jnp.float32).at[:, Cout].set(1.0)
    bcat = bcat.reshape(1, NTYPES * WPAD)
    return wx, wo, bcat


def kernel(data, edge_index_0, edge_type_0, node_type_0,
           edge_index_1, edge_type_1, node_type_1, depth,
           W_down0, b_down0, gns_down0, gnb_down0,
           W_conv0, gns_conv0, gnb_conv0,
           W_down1, b_down1, gns_down1, gnb_down1,
           W_conv1, gns_conv1, gnb_conv1):
    del depth
    N0, C0 = data.shape
    N1 = node_type_0.shape[0]
    N2 = node_type_1.shape[0]
    C1 = W_down1.shape[1]
    NT0 = W_conv0.shape[1] - C0
    NT1 = W_conv1.shape[1] - C1
    W0, W1 = 48, 80
    B = 128

    wx0, wo0, bcat0 = _prep_conv_weights(W_conv0, C0, NT0, W0)
    pooled = _sc_pool(data, NOUT=N1)
    xt0 = _tc_stage0(node_type_0.reshape(N1, 1), pooled, W_down0 / 8.0,
                     b_down0.reshape(1, C0), gns_down0.reshape(1, C0),
                     gnb_down0.reshape(1, C0), wx0, wo0, bcat0,
                     C=C0, NT=NT0, bn=2048)
    part0 = _sc_conv(xt0.reshape(N1 * NTYPES, W0), edge_index_0.reshape(2, -1, B),
                     edge_type_0.reshape(-1, B), NACC=N1, WIDTH=W0)

    wx1, wo1, bcat1 = _prep_conv_weights(W_conv1, C1, NT1, W1)
    xt1 = _tc_mid(node_type_1.reshape(N2, 1), part0[0], part0[1],
                  gns_conv0.reshape(1, C0), gnb_conv0.reshape(1, C0),
                  W_down1, b_down1.reshape(1, C1), gns_down1.reshape(1, C1),
                  gnb_down1.reshape(1, C1), wx1, wo1, bcat1,
                  C0=C0, C1=C1, NT=NT1)
    part1 = _sc_conv(xt1.reshape(N2 * NTYPES, W1), edge_index_1.reshape(2, -1, B),
                     edge_type_1.reshape(-1, B), NACC=N2, WIDTH=W1)
    out = _tc_final(part1[0], part1[1], gns_conv1.reshape(1, C1),
                    gnb_conv1.reshape(1, C1), C=C1)
    return out


# speedup vs baseline: 1.1390x; 1.1390x over previous
import functools

import jax
import jax.numpy as jnp
from jax import lax
from jax.experimental import pallas as pl
from jax.experimental.pallas import tpu as pltpu
from jax.experimental.pallas import tpu_sc as plsc

NC, NS, LANES = 2, 16, 16
NW = NC * NS
GROUPS = 8
EPS = 1e-5
NTYPES = 7


def _gn(h, gns, gnb, C):
    g = C // GROUPS
    r = lax.broadcasted_iota(jnp.int32, (C, C), 0) // g
    c = lax.broadcasted_iota(jnp.int32, (C, C), 1) // g
    mg = (r == c).astype(jnp.float32) / g
    m = jnp.dot(h, mg, preferred_element_type=jnp.float32)
    e2 = jnp.dot(h * h, mg, preferred_element_type=jnp.float32)
    v = e2 - m * m
    return (h - m) * lax.rsqrt(v + EPS) * gns + gnb


def _down(xin, w_ref, b_ref, gns_ref, gnb_ref, C):
    n8, cin = xin.shape
    xp = jnp.mean(xin.reshape(n8 // 8, 8, cin), axis=1)
    h = jnp.dot(xp, w_ref[:], preferred_element_type=jnp.float32) + b_ref[:]
    return jax.nn.gelu(_gn(h, gns_ref[:], gnb_ref[:], C))


def _table(x, nt_ref, wx_ref, wo_ref, bcat_ref, NT):
    oh = (nt_ref[:] == lax.broadcasted_iota(jnp.int32, (1, NT), 1))
    oh = oh.astype(jnp.float32)
    return (jnp.dot(x, wx_ref[:], preferred_element_type=jnp.float32)
            + jnp.dot(oh, wo_ref[:], preferred_element_type=jnp.float32)
            + bcat_ref[:])


def _tc_stage0_body(nt_ref, d_ref, w_ref, b_ref, gns_ref, gnb_ref,
                    wx_ref, wo_ref, bcat_ref, out_ref, *, C, NT):
    x = _down(d_ref[:], w_ref, b_ref, gns_ref, gnb_ref, C)
    out_ref[:] = _table(x, nt_ref, wx_ref, wo_ref, bcat_ref, NT)


def _tc_stage0(nt2, data, w, b, gns, gnb, wx, wo, bcat, *, C, NT, bn):
    N8, CIN = data.shape
    N = N8 // 8
    TW = wx.shape[1]
    return pl.pallas_call(
        functools.partial(_tc_stage0_body, C=C, NT=NT),
        grid=(N // bn,),
        in_specs=[
            pl.BlockSpec((bn, 1), lambda i: (i, 0)),
            pl.BlockSpec((bn * 8, CIN), lambda i: (i, 0)),
            pl.BlockSpec((CIN, C), lambda i: (0, 0)),
            pl.BlockSpec((1, C), lambda i: (0, 0)),
            pl.BlockSpec((1, C), lambda i: (0, 0)),
            pl.BlockSpec((1, C), lambda i: (0, 0)),
            pl.BlockSpec((C, TW), lambda i: (0, 0)),
            pl.BlockSpec((NT, TW), lambda i: (0, 0)),
            pl.BlockSpec((1, TW), lambda i: (0, 0)),
        ],
        out_specs=pl.BlockSpec((bn, TW), lambda i: (i, 0)),
        out_shape=jax.ShapeDtypeStruct((N, TW), jnp.float32),
    )(nt2, data, w, b, gns, gnb, wx, wo, bcat)


def _tc_mid_body(nt_ref, p0_ref, p1_ref, gnsc_ref, gnbc_ref,
                 w_ref, b_ref, gns_ref, gnb_ref, wx_ref, wo_ref, bcat_ref,
                 out_ref, *, C0, C1, NT):
    s = p0_ref[:, :C0 + 1] + p1_ref[:, :C0 + 1]
    deg = jnp.maximum(s[:, C0:C0 + 1], 1.0)
    agg = s[:, :C0] / deg
    x1 = jax.nn.gelu(_gn(agg, gnsc_ref[:], gnbc_ref[:], C0))
    x2 = _down(x1, w_ref, b_ref, gns_ref, gnb_ref, C1)
    out_ref[:] = _table(x2, nt_ref, wx_ref, wo_ref, bcat_ref, NT)


def _tc_mid(nt2, p0, p1, gnsc, gnbc, w, b, gns, gnb, wx, wo, bcat,
            *, C0, C1, NT):
    N1, W = p0.shape
    N2 = N1 // 8
    TW = wx.shape[1]
    return pl.pallas_call(
        functools.partial(_tc_mid_body, C0=C0, C1=C1, NT=NT),
        out_shape=jax.ShapeDtypeStruct((N2, TW), jnp.float32),
    )(nt2, p0, p1, gnsc, gnbc, w, b, gns, gnb, wx, wo, bcat)


def _tc_final_body(p0_ref, p1_ref, gns_ref, gnb_ref, out_ref, *, C):
    s = p0_ref[:, :C + 1] + p1_ref[:, :C + 1]
    deg = jnp.maximum(s[:, C:C + 1], 1.0)
    agg = s[:, :C] / deg
    out_ref[:] = jax.nn.gelu(_gn(agg, gns_ref[:], gnb_ref[:], C))


def _tc_final(p0, p1, gns, gnb, *, C):
    N, W = p0.shape
    return pl.pallas_call(
        functools.partial(_tc_final_body, C=C),
        out_shape=jax.ShapeDtypeStruct((N, C), jnp.float32),
    )(p0, p1, gns, gnb)


def _sc_conv(xt2d, ei3, et2, *, NACC, WIDTH, B=128, NBUF=8):
    NTAB, TW = xt2d.shape
    E = ei3.shape[1] * B
    ew = E // NW
    nchunk = ew // B
    nouter = nchunk // NBUF
    rps = NACC // NS
    mesh = plsc.VectorSubcoreMesh(core_axis_name="c", subcore_axis_name="s",
                                  num_cores=NC, num_subcores=NS)

    OUTW = 128

    @functools.partial(
        pl.kernel,
        out_type=jax.ShapeDtypeStruct((NC, NACC, OUTW), jnp.float32),
        mesh=mesh,
        compiler_params=pltpu.CompilerParams(use_tc_tiling_on_sc=False),
        scratch_types=[
            pltpu.VMEM((nchunk, B), jnp.int32),
            pltpu.VMEM((nchunk, B), jnp.int32),
            pltpu.VMEM((nchunk, B), jnp.int32),
            pltpu.VMEM((nchunk, B), jnp.int32),
            pltpu.VMEM((NBUF, B, WIDTH), jnp.float32),
            pltpu.VMEM_SHARED((NACC, WIDTH), jnp.float32),
            [pltpu.SemaphoreType.DMA] * NBUF,
            [pltpu.SemaphoreType.DMA] * NBUF,
        ],
    )
    def k(xt_hbm, ei_hbm, et_hbm, out_hbm,
          srcb, typb, dstb, idxb, rowsb, acc, gsem, ssem):
        cid = lax.axis_index("c")
        sid = lax.axis_index("s")
        wid = sid * NC + cid
        row0 = wid * nchunk
        xtr = xt_hbm

        pltpu.sync_copy(ei_hbm.at[0, pl.ds(row0, nchunk)], srcb)
        pltpu.sync_copy(ei_hbm.at[1, pl.ds(row0, nchunk)], dstb)
        pltpu.sync_copy(et_hbm.at[pl.ds(row0, nchunk)], typb)

        @pl.loop(0, B)
        def _(i):
            for cc in range(WIDTH // LANES):
                rowsb[0, i, pl.ds(cc * LANES, LANES)] = jnp.zeros(
                    (LANES,), jnp.float32)
        rbase = sid * rps
        for kk in range(rps // B):
            pltpu.sync_copy(rowsb.at[0], acc.at[pl.ds(rbase + kk * B, B)])

        @pl.loop(0, nchunk)
        def _(c):
            for j in range(B // LANES):
                sl = pl.ds(j * LANES, LANES)
                idxb[c, sl] = srcb[c, sl] * NTYPES + typb[c, sl]
        plsc.subcore_barrier()

        def fire_gather(c, b):
            pltpu.async_copy(xtr.at[idxb.at[c]], rowsb.at[b], gsem[b])

        def wait_gather(b):
            pltpu.make_async_copy(
                xtr.at[idxb.at[0]], rowsb.at[b], gsem[b]).wait()

        def fire_scatter(c, b):
            pltpu.async_copy(rowsb.at[b], acc.at[dstb.at[c]], ssem[b],
                             add=True)

        def wait_scatter(b):
            pltpu.make_async_copy(
                rowsb.at[b], acc.at[dstb.at[0]], ssem[b]).wait()

        for b in range(NBUF):
            fire_gather(b, b)

        @pl.loop(0, nouter - 1)
        def _(i):
            c0 = i * NBUF
            for b in range(NBUF):
                wait_gather(b)
                fire_scatter(c0 + b, b)
            for b in range(NBUF):
                wait_scatter(b)
                fire_gather(c0 + NBUF + b, b)

        c0 = (nouter - 1) * NBUF
        for b in range(NBUF):
            wait_gather(b)
            fire_scatter(c0 + b, b)
        for b in range(NBUF):
            wait_scatter(b)

        plsc.subcore_barrier()
        for kk in range(rps // B):
            r0 = rbase + kk * B
            pltpu.sync_copy(acc.at[pl.ds(r0, B)], rowsb.at[0])
            pltpu.sync_copy(rowsb.at[0],
                            out_hbm.at[cid, pl.ds(r0, B), pl.ds(0, WIDTH)])

    return k(xt2d, ei3, et2)


def _prep_conv_weights(W_conv, C, NT, WPAD):
    Cout = W_conv.shape[2]
    wx = jnp.transpose(W_conv[:, :C, :], (1, 0, 2))
    wx = jnp.pad(wx, ((0, 0), (0, 0), (0, WPAD - Cout)))
    wx = wx.reshape(C, NTYPES * WPAD)
    wo = jnp.transpose(W_conv[:, C:, :], (1, 0, 2))
    wo = jnp.pad(wo, ((0, 0), (0, 0), (0, WPAD - Cout)))
    wo = wo.reshape(NT, NTYPES * WPAD)
    bcat = jnp.zeros((NTYPES, WPAD), jnp.float32).at[:, Cout].set(1.0)
    bcat = bcat.reshape(1, NTYPES * WPAD)
    return wx, wo, bcat


def kernel(data, edge_index_0, edge_type_0, node_type_0,
           edge_index_1, edge_type_1, node_type_1, depth,
           W_down0, b_down0, gns_down0, gnb_down0,
           W_conv0, gns_conv0, gnb_conv0,
           W_down1, b_down1, gns_down1, gnb_down1,
           W_conv1, gns_conv1, gnb_conv1):
    del depth
    N0, C0 = data.shape
    N1 = node_type_0.shape[0]
    N2 = node_type_1.shape[0]
    C1 = W_down1.shape[1]
    NT0 = W_conv0.shape[1] - C0
    NT1 = W_conv1.shape[1] - C1
    W0, W1 = 48, 80
    B = 128

    wx0, wo0, bcat0 = _prep_conv_weights(W_conv0, C0, NT0, W0)
    xt0 = _tc_stage0(node_type_0.reshape(N1, 1), data, W_down0,
                     b_down0.reshape(1, C0), gns_down0.reshape(1, C0),
                     gnb_down0.reshape(1, C0), wx0, wo0, bcat0,
                     C=C0, NT=NT0, bn=2048)
    part0 = _sc_conv(xt0.reshape(N1 * NTYPES, W0), edge_index_0.reshape(2, -1, B),
                     edge_type_0.reshape(-1, B), NACC=N1, WIDTH=W0)

    wx1, wo1, bcat1 = _prep_conv_weights(W_conv1, C1, NT1, W1)
    xt1 = _tc_mid(node_type_1.reshape(N2, 1), part0[0], part0[1],
                  gns_conv0.reshape(1, C0), gnb_conv0.reshape(1, C0),
                  W_down1, b_down1.reshape(1, C1), gns_down1.reshape(1, C1),
                  gnb_down1.reshape(1, C1), wx1, wo1, bcat1,
                  C0=C0, C1=C1, NT=NT1)
    part1 = _sc_conv(xt1.reshape(N2 * NTYPES, W1), edge_index_1.reshape(2, -1, B),
                     edge_type_1.reshape(-1, B), NACC=N2, WIDTH=W1)
    out = _tc_final(part1[0], part1[1], gns_conv1.reshape(1, C1),
                    gnb_conv1.reshape(1, C1), C=C1)
    return out


# speedup vs baseline: 1.1522x; 1.0116x over previous
import functools

import jax
import jax.numpy as jnp
from jax import lax
from jax.experimental import pallas as pl
from jax.experimental.pallas import tpu as pltpu
from jax.experimental.pallas import tpu_sc as plsc

NC, NS, LANES = 2, 16, 16
NW = NC * NS
GROUPS = 8
EPS = 1e-5
NTYPES = 7


def _gn(h, gns, gnb, C):
    g = C // GROUPS
    r = lax.broadcasted_iota(jnp.int32, (C, C), 0) // g
    c = lax.broadcasted_iota(jnp.int32, (C, C), 1) // g
    mg = (r == c).astype(jnp.float32) / g
    m = jnp.dot(h, mg, preferred_element_type=jnp.float32)
    e2 = jnp.dot(h * h, mg, preferred_element_type=jnp.float32)
    v = e2 - m * m
    return (h - m) * lax.rsqrt(v + EPS) * gns + gnb


def _down(xin, w_ref, b_ref, gns_ref, gnb_ref, C):
    n8, cin = xin.shape
    xp = jnp.mean(xin.reshape(n8 // 8, 8, cin), axis=1)
    h = jnp.dot(xp, w_ref[:], preferred_element_type=jnp.float32) + b_ref[:]
    return jax.nn.gelu(_gn(h, gns_ref[:], gnb_ref[:], C))


def _table(x, nt_ref, wx_ref, wo_ref, bcat_ref, NT):
    oh = (nt_ref[:] == lax.broadcasted_iota(jnp.int32, (1, NT), 1))
    oh = oh.astype(jnp.float32)
    return (jnp.dot(x, wx_ref[:], preferred_element_type=jnp.float32)
            + jnp.dot(oh, wo_ref[:], preferred_element_type=jnp.float32)
            + bcat_ref[:])


def _tc_stage0_body(nt_ref, d_ref, w_ref, b_ref, gns_ref, gnb_ref,
                    wx_ref, wo_ref, bcat_ref, out_ref, *, C, NT):
    x = _down(d_ref[:], w_ref, b_ref, gns_ref, gnb_ref, C)
    out_ref[:] = _table(x, nt_ref, wx_ref, wo_ref, bcat_ref, NT)


def _tc_stage0(nt2, data, w, b, gns, gnb, wx, wo, bcat, *, C, NT, bn):
    N8, CIN = data.shape
    N = N8 // 8
    TW = wx.shape[1]
    return pl.pallas_call(
        functools.partial(_tc_stage0_body, C=C, NT=NT),
        grid=(N // bn,),
        in_specs=[
            pl.BlockSpec((bn, 1), lambda i: (i, 0)),
            pl.BlockSpec((bn * 8, CIN), lambda i: (i, 0)),
            pl.BlockSpec((CIN, C), lambda i: (0, 0)),
            pl.BlockSpec((1, C), lambda i: (0, 0)),
            pl.BlockSpec((1, C), lambda i: (0, 0)),
            pl.BlockSpec((1, C), lambda i: (0, 0)),
            pl.BlockSpec((C, TW), lambda i: (0, 0)),
            pl.BlockSpec((NT, TW), lambda i: (0, 0)),
            pl.BlockSpec((1, TW), lambda i: (0, 0)),
        ],
        out_specs=pl.BlockSpec((bn, TW), lambda i: (i, 0)),
        out_shape=jax.ShapeDtypeStruct((N, TW), jnp.float32),
    )(nt2, data, w, b, gns, gnb, wx, wo, bcat)


def _tc_mid_body(nt_ref, p0_ref, p1_ref, gnsc_ref, gnbc_ref,
                 w_ref, b_ref, gns_ref, gnb_ref, wx_ref, wo_ref, bcat_ref,
                 out_ref, *, C0, C1, NT):
    s = p0_ref[:, :C0 + 1] + p1_ref[:, :C0 + 1]
    deg = jnp.maximum(s[:, C0:C0 + 1], 1.0)
    agg = s[:, :C0] / deg
    x1 = jax.nn.gelu(_gn(agg, gnsc_ref[:], gnbc_ref[:], C0))
    x2 = _down(x1, w_ref, b_ref, gns_ref, gnb_ref, C1)
    out_ref[:] = _table(x2, nt_ref, wx_ref, wo_ref, bcat_ref, NT)


def _tc_mid(nt2, p0, p1, gnsc, gnbc, w, b, gns, gnb, wx, wo, bcat,
            *, C0, C1, NT, bn=4096):
    N1, W = p0.shape
    N2 = N1 // 8
    TW = wx.shape[1]
    PW = W
    bo = bn // 8
    return pl.pallas_call(
        functools.partial(_tc_mid_body, C0=C0, C1=C1, NT=NT),
        grid=(N1 // bn,),
        in_specs=[
            pl.BlockSpec((bo, 1), lambda i: (i, 0)),
            pl.BlockSpec((bn, PW), lambda i: (i, 0)),
            pl.BlockSpec((bn, PW), lambda i: (i, 0)),
            pl.BlockSpec((1, C0), lambda i: (0, 0)),
            pl.BlockSpec((1, C0), lambda i: (0, 0)),
            pl.BlockSpec((C0, C1), lambda i: (0, 0)),
            pl.BlockSpec((1, C1), lambda i: (0, 0)),
            pl.BlockSpec((1, C1), lambda i: (0, 0)),
            pl.BlockSpec((1, C1), lambda i: (0, 0)),
            pl.BlockSpec((C1, TW), lambda i: (0, 0)),
            pl.BlockSpec((NT, TW), lambda i: (0, 0)),
            pl.BlockSpec((1, TW), lambda i: (0, 0)),
        ],
        out_specs=pl.BlockSpec((bo, TW), lambda i: (i, 0)),
        out_shape=jax.ShapeDtypeStruct((N2, TW), jnp.float32),
    )(nt2, p0, p1, gnsc, gnbc, w, b, gns, gnb, wx, wo, bcat)


def _tc_final_body(p0_ref, p1_ref, gns_ref, gnb_ref, out_ref, *, C):
    s = p0_ref[:, :C + 1] + p1_ref[:, :C + 1]
    deg = jnp.maximum(s[:, C:C + 1], 1.0)
    agg = s[:, :C] / deg
    out_ref[:] = jax.nn.gelu(_gn(agg, gns_ref[:], gnb_ref[:], C))


def _tc_final(p0, p1, gns, gnb, *, C):
    N, W = p0.shape
    return pl.pallas_call(
        functools.partial(_tc_final_body, C=C),
        out_shape=jax.ShapeDtypeStruct((N, C), jnp.float32),
    )(p0, p1, gns, gnb)


def _sc_conv(xt2d, ei3, et2, *, NACC, WIDTH, B=128, NBUF=8):
    NTAB, TW = xt2d.shape
    E = ei3.shape[1] * B
    ew = E // NW
    nchunk = ew // B
    nouter = nchunk // NBUF
    rps = NACC // NS
    mesh = plsc.VectorSubcoreMesh(core_axis_name="c", subcore_axis_name="s",
                                  num_cores=NC, num_subcores=NS)

    OUTW = 128

    @functools.partial(
        pl.kernel,
        out_type=jax.ShapeDtypeStruct((NC, NACC, OUTW), jnp.float32),
        mesh=mesh,
        compiler_params=pltpu.CompilerParams(use_tc_tiling_on_sc=False),
        scratch_types=[
            pltpu.VMEM((nchunk, B), jnp.int32),
            pltpu.VMEM((nchunk, B), jnp.int32),
            pltpu.VMEM((nchunk, B), jnp.int32),
            pltpu.VMEM((nchunk, B), jnp.int32),
            pltpu.VMEM((NBUF, B, WIDTH), jnp.float32),
            pltpu.VMEM_SHARED((NACC, WIDTH), jnp.float32),
            [pltpu.SemaphoreType.DMA] * NBUF,
            [pltpu.SemaphoreType.DMA] * NBUF,
        ],
    )
    def k(xt_hbm, ei_hbm, et_hbm, out_hbm,
          srcb, typb, dstb, idxb, rowsb, acc, gsem, ssem):
        cid = lax.axis_index("c")
        sid = lax.axis_index("s")
        wid = sid * NC + cid
        row0 = wid * nchunk
        xtr = xt_hbm

        pltpu.sync_copy(ei_hbm.at[0, pl.ds(row0, nchunk)], srcb)
        pltpu.sync_copy(ei_hbm.at[1, pl.ds(row0, nchunk)], dstb)
        pltpu.sync_copy(et_hbm.at[pl.ds(row0, nchunk)], typb)

        @pl.loop(0, B)
        def _(i):
            for cc in range(WIDTH // LANES):
                rowsb[0, i, pl.ds(cc * LANES, LANES)] = jnp.zeros(
                    (LANES,), jnp.float32)
        rbase = sid * rps
        for kk in range(rps // B):
            pltpu.sync_copy(rowsb.at[0], acc.at[pl.ds(rbase + kk * B, B)])

        @pl.loop(0, nchunk)
        def _(c):
            for j in range(B // LANES):
                sl = pl.ds(j * LANES, LANES)
                idxb[c, sl] = srcb[c, sl] * NTYPES + typb[c, sl]
        plsc.subcore_barrier()

        def fire_gather(c, b):
            pltpu.async_copy(xtr.at[idxb.at[c]], rowsb.at[b], gsem[b])

        def wait_gather(b):
            pltpu.make_async_copy(
                xtr.at[idxb.at[0]], rowsb.at[b], gsem[b]).wait()

        def fire_scatter(c, b):
            pltpu.async_copy(rowsb.at[b], acc.at[dstb.at[c]], ssem[b],
                             add=True)

        def wait_scatter(b):
            pltpu.make_async_copy(
                rowsb.at[b], acc.at[dstb.at[0]], ssem[b]).wait()

        for b in range(NBUF):
            fire_gather(b, b)

        @pl.loop(0, nouter - 1)
        def _(i):
            c0 = i * NBUF
            for b in range(NBUF):
                wait_gather(b)
                fire_scatter(c0 + b, b)
            for b in range(NBUF):
                wait_scatter(b)
                fire_gather(c0 + NBUF + b, b)

        c0 = (nouter - 1) * NBUF
        for b in range(NBUF):
            wait_gather(b)
            fire_scatter(c0 + b, b)
        for b in range(NBUF):
            wait_scatter(b)

        plsc.subcore_barrier()
        for kk in range(rps // B):
            r0 = rbase + kk * B
            pltpu.sync_copy(acc.at[pl.ds(r0, B)], rowsb.at[0])
            pltpu.sync_copy(rowsb.at[0],
                            out_hbm.at[cid, pl.ds(r0, B), pl.ds(0, WIDTH)])

    return k(xt2d, ei3, et2)


def _prep_conv_weights(W_conv, C, NT, WPAD):
    Cout = W_conv.shape[2]
    wx = jnp.transpose(W_conv[:, :C, :], (1, 0, 2))
    wx = jnp.pad(wx, ((0, 0), (0, 0), (0, WPAD - Cout)))
    wx = wx.reshape(C, NTYPES * WPAD)
    wo = jnp.transpose(W_conv[:, C:, :], (1, 0, 2))
    wo = jnp.pad(wo, ((0, 0), (0, 0), (0, WPAD - Cout)))
    wo = wo.reshape(NT, NTYPES * WPAD)
    bcat = jnp.zeros((NTYPES, WPAD), jnp.float32).at[:, Cout].set(1.0)
    bcat = bcat.reshape(1, NTYPES * WPAD)
    return wx, wo, bcat


def kernel(data, edge_index_0, edge_type_0, node_type_0,
           edge_index_1, edge_type_1, node_type_1, depth,
           W_down0, b_down0, gns_down0, gnb_down0,
           W_conv0, gns_conv0, gnb_conv0,
           W_down1, b_down1, gns_down1, gnb_down1,
           W_conv1, gns_conv1, gnb_conv1):
    del depth
    N0, C0 = data.shape
    N1 = node_type_0.shape[0]
    N2 = node_type_1.shape[0]
    C1 = W_down1.shape[1]
    NT0 = W_conv0.shape[1] - C0
    NT1 = W_conv1.shape[1] - C1
    W0, W1 = 48, 80
    B = 128

    wx0, wo0, bcat0 = _prep_conv_weights(W_conv0, C0, NT0, W0)
    xt0 = _tc_stage0(node_type_0.reshape(N1, 1), data, W_down0,
                     b_down0.reshape(1, C0), gns_down0.reshape(1, C0),
                     gnb_down0.reshape(1, C0), wx0, wo0, bcat0,
                     C=C0, NT=NT0, bn=2048)
    part0 = _sc_conv(xt0.reshape(N1 * NTYPES, W0), edge_index_0.reshape(2, -1, B),
                     edge_type_0.reshape(-1, B), NACC=N1, WIDTH=W0)

    wx1, wo1, bcat1 = _prep_conv_weights(W_conv1, C1, NT1, W1)
    xt1 = _tc_mid(node_type_1.reshape(N2, 1), part0[0], part0[1],
                  gns_conv0.reshape(1, C0), gnb_conv0.reshape(1, C0),
                  W_down1, b_down1.reshape(1, C1), gns_down1.reshape(1, C1),
                  gnb_down1.reshape(1, C1), wx1, wo1, bcat1,
                  C0=C0, C1=C1, NT=NT1)
    part1 = _sc_conv(xt1.reshape(N2 * NTYPES, W1), edge_index_1.reshape(2, -1, B),
                     edge_type_1.reshape(-1, B), NACC=N2, WIDTH=W1)
    out = _tc_final(part1[0], part1[1], gns_conv1.reshape(1, C1),
                    gnb_conv1.reshape(1, C1), C=C1)
    return out


# speedup vs baseline: 1.1552x; 1.0025x over previous
import functools

import jax
import jax.numpy as jnp
from jax import lax
from jax.experimental import pallas as pl
from jax.experimental.pallas import tpu as pltpu
from jax.experimental.pallas import tpu_sc as plsc

NC, NS, LANES = 2, 16, 16
NW = NC * NS
GROUPS = 8
EPS = 1e-5
NTYPES = 7


def _gn(h, gns, gnb, C):
    g = C // GROUPS
    r = lax.broadcasted_iota(jnp.int32, (C, C), 0) // g
    c = lax.broadcasted_iota(jnp.int32, (C, C), 1) // g
    mg = (r == c).astype(jnp.float32) / g
    m = jnp.dot(h, mg, preferred_element_type=jnp.float32)
    d = h - m
    v = jnp.dot(d * d, mg, preferred_element_type=jnp.float32)
    return d * lax.rsqrt(v + EPS) * gns + gnb


def _down(xin, w_ref, b_ref, gns_ref, gnb_ref, C):
    n8, cin = xin.shape
    xp = jnp.mean(xin.reshape(n8 // 8, 8, cin), axis=1)
    h = jnp.dot(xp, w_ref[:], preferred_element_type=jnp.float32) + b_ref[:]
    return jax.nn.gelu(_gn(h, gns_ref[:], gnb_ref[:], C))


def _table(x, nt_ref, wx_ref, wo_ref, bcat_ref, NT):
    oh = (nt_ref[:] == lax.broadcasted_iota(jnp.int32, (1, NT), 1))
    oh = oh.astype(jnp.float32)
    return (jnp.dot(x, wx_ref[:], preferred_element_type=jnp.float32)
            + jnp.dot(oh, wo_ref[:], preferred_element_type=jnp.float32)
            + bcat_ref[:])


def _tc_stage0_body(nt_ref, d_ref, w_ref, b_ref, gns_ref, gnb_ref,
                    wx_ref, wo_ref, bcat_ref, out_ref, *, C, NT):
    x = _down(d_ref[:], w_ref, b_ref, gns_ref, gnb_ref, C)
    out_ref[:] = _table(x, nt_ref, wx_ref, wo_ref, bcat_ref, NT)


def _tc_stage0(nt2, data, w, b, gns, gnb, wx, wo, bcat, *, C, NT, bn):
    N8, CIN = data.shape
    N = N8 // 8
    TW = wx.shape[1]
    return pl.pallas_call(
        functools.partial(_tc_stage0_body, C=C, NT=NT),
        grid=(N // bn,),
        in_specs=[
            pl.BlockSpec((bn, 1), lambda i: (i, 0)),
            pl.BlockSpec((bn * 8, CIN), lambda i: (i, 0)),
            pl.BlockSpec((CIN, C), lambda i: (0, 0)),
            pl.BlockSpec((1, C), lambda i: (0, 0)),
            pl.BlockSpec((1, C), lambda i: (0, 0)),
            pl.BlockSpec((1, C), lambda i: (0, 0)),
            pl.BlockSpec((C, TW), lambda i: (0, 0)),
            pl.BlockSpec((NT, TW), lambda i: (0, 0)),
            pl.BlockSpec((1, TW), lambda i: (0, 0)),
        ],
        out_specs=pl.BlockSpec((bn, TW), lambda i: (i, 0)),
        out_shape=jax.ShapeDtypeStruct((N, TW), jnp.float32),
    )(nt2, data, w, b, gns, gnb, wx, wo, bcat)


def _tc_mid_body(nt_ref, p0_ref, p1_ref, gnsc_ref, gnbc_ref,
                 w_ref, b_ref, gns_ref, gnb_ref, wx_ref, wo_ref, bcat_ref,
                 out_ref, *, C0, C1, NT):
    s = p0_ref[:, :C0 + 1] + p1_ref[:, :C0 + 1]
    deg = jnp.maximum(s[:, C0:C0 + 1], 1.0)
    agg = s[:, :C0] / deg
    x1 = jax.nn.gelu(_gn(agg, gnsc_ref[:], gnbc_ref[:], C0))
    x2 = _down(x1, w_ref, b_ref, gns_ref, gnb_ref, C1)
    out_ref[:] = _table(x2, nt_ref, wx_ref, wo_ref, bcat_ref, NT)


def _tc_mid(nt2, p0, p1, gnsc, gnbc, w, b, gns, gnb, wx, wo, bcat,
            *, C0, C1, NT, bn=4096):
    N1, W = p0.shape
    N2 = N1 // 8
    TW = wx.shape[1]
    PW = W
    bo = bn // 8
    return pl.pallas_call(
        functools.partial(_tc_mid_body, C0=C0, C1=C1, NT=NT),
        grid=(N1 // bn,),
        in_specs=[
            pl.BlockSpec((bo, 1), lambda i: (i, 0)),
            pl.BlockSpec((bn, PW), lambda i: (i, 0)),
            pl.BlockSpec((bn, PW), lambda i: (i, 0)),
            pl.BlockSpec((1, C0), lambda i: (0, 0)),
            pl.BlockSpec((1, C0), lambda i: (0, 0)),
            pl.BlockSpec((C0, C1), lambda i: (0, 0)),
            pl.BlockSpec((1, C1), lambda i: (0, 0)),
            pl.BlockSpec((1, C1), lambda i: (0, 0)),
            pl.BlockSpec((1, C1), lambda i: (0, 0)),
            pl.BlockSpec((C1, TW), lambda i: (0, 0)),
            pl.BlockSpec((NT, TW), lambda i: (0, 0)),
            pl.BlockSpec((1, TW), lambda i: (0, 0)),
        ],
        out_specs=pl.BlockSpec((bo, TW), lambda i: (i, 0)),
        out_shape=jax.ShapeDtypeStruct((N2, TW), jnp.float32),
    )(nt2, p0, p1, gnsc, gnbc, w, b, gns, gnb, wx, wo, bcat)


def _tc_final_body(p0_ref, p1_ref, gns_ref, gnb_ref, out_ref, *, C):
    s = p0_ref[:, :C + 1] + p1_ref[:, :C + 1]
    deg = jnp.maximum(s[:, C:C + 1], 1.0)
    agg = s[:, :C] / deg
    out_ref[:] = jax.nn.gelu(_gn(agg, gns_ref[:], gnb_ref[:], C))


def _tc_final(p0, p1, gns, gnb, *, C):
    N, W = p0.shape
    return pl.pallas_call(
        functools.partial(_tc_final_body, C=C),
        out_shape=jax.ShapeDtypeStruct((N, C), jnp.float32),
    )(p0, p1, gns, gnb)


def _sc_conv(xt2d, ei3, et2, *, NACC, WIDTH, B=128, NBUF=8):
    NTAB, TW = xt2d.shape
    E = ei3.shape[1] * B
    ew = E // NW
    nchunk = ew // B
    nouter = nchunk // NBUF
    rps = NACC // NS
    mesh = plsc.VectorSubcoreMesh(core_axis_name="c", subcore_axis_name="s",
                                  num_cores=NC, num_subcores=NS)

    OUTW = 128

    @functools.partial(
        pl.kernel,
        out_type=jax.ShapeDtypeStruct((NC, NACC, OUTW), jnp.float32),
        mesh=mesh,
        compiler_params=pltpu.CompilerParams(use_tc_tiling_on_sc=False),
        scratch_types=[
            pltpu.VMEM((nchunk, B), jnp.int32),
            pltpu.VMEM((nchunk, B), jnp.int32),
            pltpu.VMEM((nchunk, B), jnp.int32),
            pltpu.VMEM((nchunk, B), jnp.int32),
            pltpu.VMEM((NBUF, B, WIDTH), jnp.float32),
            pltpu.VMEM_SHARED((NACC, WIDTH), jnp.float32),
            [pltpu.SemaphoreType.DMA] * NBUF,
            [pltpu.SemaphoreType.DMA] * NBUF,
        ],
    )
    def k(xt_hbm, ei_hbm, et_hbm, out_hbm,
          srcb, typb, dstb, idxb, rowsb, acc, gsem, ssem):
        cid = lax.axis_index("c")
        sid = lax.axis_index("s")
        wid = sid * NC + cid
        row0 = wid * nchunk
        xtr = xt_hbm

        pltpu.sync_copy(ei_hbm.at[0, pl.ds(row0, nchunk)], srcb)
        pltpu.sync_copy(ei_hbm.at[1, pl.ds(row0, nchunk)], dstb)
        pltpu.sync_copy(et_hbm.at[pl.ds(row0, nchunk)], typb)

        @pl.loop(0, B)
        def _(i):
            for cc in range(WIDTH // LANES):
                rowsb[0, i, pl.ds(cc * LANES, LANES)] = jnp.zeros(
                    (LANES,), jnp.float32)
        rbase = sid * rps
        for kk in range(rps // B):
            pltpu.sync_copy(rowsb.at[0], acc.at[pl.ds(rbase + kk * B, B)])

        @pl.loop(0, nchunk)
        def _(c):
            for j in range(B // LANES):
                sl = pl.ds(j * LANES, LANES)
                idxb[c, sl] = srcb[c, sl] * NTYPES + typb[c, sl]
        plsc.subcore_barrier()

        def fire_gather(c, b):
            pltpu.async_copy(xtr.at[idxb.at[c]], rowsb.at[b], gsem[b])

        def wait_gather(b):
            pltpu.make_async_copy(
                xtr.at[idxb.at[0]], rowsb.at[b], gsem[b]).wait()

        def fire_scatter(c, b):
            pltpu.async_copy(rowsb.at[b], acc.at[dstb.at[c]], ssem[b],
                             add=True)

        def wait_scatter(b):
            pltpu.make_async_copy(
                rowsb.at[b], acc.at[dstb.at[0]], ssem[b]).wait()

        for b in range(NBUF):
            fire_gather(b, b)

        @pl.loop(0, nouter - 1)
        def _(i):
            c0 = i * NBUF
            for b in range(NBUF):
                wait_gather(b)
                fire_scatter(c0 + b, b)
            for b in range(NBUF):
                wait_scatter(b)
                fire_gather(c0 + NBUF + b, b)

        c0 = (nouter - 1) * NBUF
        for b in range(NBUF):
            wait_gather(b)
            fire_scatter(c0 + b, b)
        for b in range(NBUF):
            wait_scatter(b)

        plsc.subcore_barrier()
        for kk in range(rps // B):
            r0 = rbase + kk * B
            pltpu.sync_copy(acc.at[pl.ds(r0, B)], rowsb.at[0])
            pltpu.sync_copy(rowsb.at[0],
                            out_hbm.at[cid, pl.ds(r0, B), pl.ds(0, WIDTH)])

    return k(xt2d, ei3, et2)


def _prep_conv_weights(W_conv, C, NT, WPAD):
    Cout = W_conv.shape[2]
    wx = jnp.transpose(W_conv[:, :C, :], (1, 0, 2))
    wx = jnp.pad(wx, ((0, 0), (0, 0), (0, WPAD - Cout)))
    wx = wx.reshape(C, NTYPES * WPAD)
    wo = jnp.transpose(W_conv[:, C:, :], (1, 0, 2))
    wo = jnp.pad(wo, ((0, 0), (0, 0), (0, WPAD - Cout)))
    wo = wo.reshape(NT, NTYPES * WPAD)
    bcat = jnp.zeros((NTYPES, WPAD), jnp.float32).at[:, Cout].set(1.0)
    bcat = bcat.reshape(1, NTYPES * WPAD)
    return wx, wo, bcat


def kernel(data, edge_index_0, edge_type_0, node_type_0,
           edge_index_1, edge_type_1, node_type_1, depth,
           W_down0, b_down0, gns_down0, gnb_down0,
           W_conv0, gns_conv0, gnb_conv0,
           W_down1, b_down1, gns_down1, gnb_down1,
           W_conv1, gns_conv1, gnb_conv1):
    del depth
    N0, C0 = data.shape
    N1 = node_type_0.shape[0]
    N2 = node_type_1.shape[0]
    C1 = W_down1.shape[1]
    NT0 = W_conv0.shape[1] - C0
    NT1 = W_conv1.shape[1] - C1
    W0, W1 = 48, 80
    B = 128

    wx0, wo0, bcat0 = _prep_conv_weights(W_conv0, C0, NT0, W0)
    xt0 = _tc_stage0(node_type_0.reshape(N1, 1), data, W_down0,
                     b_down0.reshape(1, C0), gns_down0.reshape(1, C0),
                     gnb_down0.reshape(1, C0), wx0, wo0, bcat0,
                     C=C0, NT=NT0, bn=2048)
    part0 = _sc_conv(xt0.reshape(N1 * NTYPES, W0), edge_index_0.reshape(2, -1, B),
                     edge_type_0.reshape(-1, B), NACC=N1, WIDTH=W0)

    wx1, wo1, bcat1 = _prep_conv_weights(W_conv1, C1, NT1, W1)
    xt1 = _tc_mid(node_type_1.reshape(N2, 1), part0[0], part0[1],
                  gns_conv0.reshape(1, C0), gnb_conv0.reshape(1, C0),
                  W_down1, b_down1.reshape(1, C1), gns_down1.reshape(1, C1),
                  gnb_down1.reshape(1, C1), wx1, wo1, bcat1,
                  C0=C0, C1=C1, NT=NT1)
    part1 = _sc_conv(xt1.reshape(N2 * NTYPES, W1), edge_index_1.reshape(2, -1, B),
                     edge_type_1.reshape(-1, B), NACC=N2, WIDTH=W1)
    out = _tc_final(part1[0], part1[1], gns_conv1.reshape(1, C1),
                    gnb_conv1.reshape(1, C1), C=C1)
    return out


# speedup vs baseline: 1.1878x; 1.0282x over previous
import functools

import jax
import jax.numpy as jnp
from jax import lax
from jax.experimental import pallas as pl
from jax.experimental.pallas import tpu as pltpu
from jax.experimental.pallas import tpu_sc as plsc

NC, NS, LANES = 2, 16, 16
NW = NC * NS
GROUPS = 8
EPS = 1e-5
NTYPES = 7


def _gn(h, gns, gnb, C):
    g = C // GROUPS
    r = lax.broadcasted_iota(jnp.int32, (C, C), 0) // g
    c = lax.broadcasted_iota(jnp.int32, (C, C), 1) // g
    mg = (r == c).astype(jnp.float32) / g
    m = jnp.dot(h, mg, preferred_element_type=jnp.float32)
    d = h - m
    v = jnp.dot(d * d, mg, preferred_element_type=jnp.float32)
    return d * lax.rsqrt(v + EPS) * gns + gnb


def _down(xin, w_ref, b_ref, gns_ref, gnb_ref, C):
    n8, cin = xin.shape
    xp = jnp.mean(xin.reshape(n8 // 8, 8, cin), axis=1)
    h = jnp.dot(xp, w_ref[:], preferred_element_type=jnp.float32) + b_ref[:]
    return jax.nn.gelu(_gn(h, gns_ref[:], gnb_ref[:], C))


def _table(x, nt_ref, wx_ref, wo_ref, bcat_ref, NT):
    oh = (nt_ref[:] == lax.broadcasted_iota(jnp.int32, (1, NT), 1))
    oh = oh.astype(jnp.float32)
    return (jnp.dot(x, wx_ref[:], preferred_element_type=jnp.float32)
            + jnp.dot(oh, wo_ref[:], preferred_element_type=jnp.float32)
            + bcat_ref[:])


def _tc_stage0_body(nt_ref, d_ref, w_ref, b_ref, gns_ref, gnb_ref,
                    wx_ref, wo_ref, bcat_ref, out_ref, *, C, NT):
    x = _down(d_ref[:], w_ref, b_ref, gns_ref, gnb_ref, C)
    out_ref[:] = _table(x, nt_ref, wx_ref, wo_ref, bcat_ref, NT)


def _tc_stage0(nt2, data, w, b, gns, gnb, wx, wo, bcat, *, C, NT, bn):
    N8, CIN = data.shape
    N = N8 // 8
    TW = wx.shape[1]
    return pl.pallas_call(
        functools.partial(_tc_stage0_body, C=C, NT=NT),
        grid=(N // bn,),
        in_specs=[
            pl.BlockSpec((bn, 1), lambda i: (i, 0)),
            pl.BlockSpec((bn * 8, CIN), lambda i: (i, 0)),
            pl.BlockSpec((CIN, C), lambda i: (0, 0)),
            pl.BlockSpec((1, C), lambda i: (0, 0)),
            pl.BlockSpec((1, C), lambda i: (0, 0)),
            pl.BlockSpec((1, C), lambda i: (0, 0)),
            pl.BlockSpec((C, TW), lambda i: (0, 0)),
            pl.BlockSpec((NT, TW), lambda i: (0, 0)),
            pl.BlockSpec((1, TW), lambda i: (0, 0)),
        ],
        out_specs=pl.BlockSpec((bn, TW), lambda i: (i, 0)),
        out_shape=jax.ShapeDtypeStruct((N, TW), jnp.float32),
    )(nt2, data, w, b, gns, gnb, wx, wo, bcat)


def _tc_mid_body(nt_ref, p0_ref, p1_ref, gnsc_ref, gnbc_ref,
                 w_ref, b_ref, gns_ref, gnb_ref, wx_ref, wo_ref, bcat_ref,
                 out_ref, *, C0, C1, NT):
    s = p0_ref[:, :C0 + 1] + p1_ref[:, :C0 + 1]
    deg = jnp.maximum(s[:, C0:C0 + 1], 1.0)
    agg = s[:, :C0] / deg
    x1 = jax.nn.gelu(_gn(agg, gnsc_ref[:], gnbc_ref[:], C0))
    x2 = _down(x1, w_ref, b_ref, gns_ref, gnb_ref, C1)
    out_ref[:] = _table(x2, nt_ref, wx_ref, wo_ref, bcat_ref, NT)


def _tc_mid(nt2, p0, p1, gnsc, gnbc, w, b, gns, gnb, wx, wo, bcat,
            *, C0, C1, NT, bn=4096):
    N1, W = p0.shape
    N2 = N1 // 8
    TW = wx.shape[1]
    PW = W
    bo = bn // 8
    return pl.pallas_call(
        functools.partial(_tc_mid_body, C0=C0, C1=C1, NT=NT),
        grid=(N1 // bn,),
        in_specs=[
            pl.BlockSpec((bo, 1), lambda i: (i, 0)),
            pl.BlockSpec((bn, PW), lambda i: (i, 0)),
            pl.BlockSpec((bn, PW), lambda i: (i, 0)),
            pl.BlockSpec((1, C0), lambda i: (0, 0)),
            pl.BlockSpec((1, C0), lambda i: (0, 0)),
            pl.BlockSpec((C0, C1), lambda i: (0, 0)),
            pl.BlockSpec((1, C1), lambda i: (0, 0)),
            pl.BlockSpec((1, C1), lambda i: (0, 0)),
            pl.BlockSpec((1, C1), lambda i: (0, 0)),
            pl.BlockSpec((C1, TW), lambda i: (0, 0)),
            pl.BlockSpec((NT, TW), lambda i: (0, 0)),
            pl.BlockSpec((1, TW), lambda i: (0, 0)),
        ],
        out_specs=pl.BlockSpec((bo, TW), lambda i: (i, 0)),
        out_shape=jax.ShapeDtypeStruct((N2, TW), jnp.float32),
    )(nt2, p0, p1, gnsc, gnbc, w, b, gns, gnb, wx, wo, bcat)


def _tc_final_body(p0_ref, p1_ref, gns_ref, gnb_ref, out_ref, *, C):
    s = p0_ref[:, :C + 1] + p1_ref[:, :C + 1]
    deg = jnp.maximum(s[:, C:C + 1], 1.0)
    agg = s[:, :C] / deg
    out_ref[:] = jax.nn.gelu(_gn(agg, gns_ref[:], gnb_ref[:], C))


def _tc_final(p0, p1, gns, gnb, *, C):
    N, W = p0.shape
    return pl.pallas_call(
        functools.partial(_tc_final_body, C=C),
        out_shape=jax.ShapeDtypeStruct((N, C), jnp.float32),
    )(p0, p1, gns, gnb)


def _sc_conv(xt2d, ei3, et2, *, NACC, WIDTH, B=128, NBUF=8):
    NTAB, TW = xt2d.shape
    E = ei3.shape[1] * B
    ew = E // NW
    nchunk = ew // B
    nouter = nchunk // NBUF
    rps = NACC // NS
    mesh = plsc.VectorSubcoreMesh(core_axis_name="c", subcore_axis_name="s",
                                  num_cores=NC, num_subcores=NS)

    OUTW = 128

    @functools.partial(
        pl.kernel,
        out_type=jax.ShapeDtypeStruct((NC, NACC, OUTW), jnp.float32),
        mesh=mesh,
        compiler_params=pltpu.CompilerParams(use_tc_tiling_on_sc=False),
        scratch_types=[
            pltpu.VMEM((nchunk, B), jnp.int32),
            pltpu.VMEM((nchunk, B), jnp.int32),
            pltpu.VMEM((nchunk, B), jnp.int32),
            pltpu.VMEM((nchunk, B), jnp.int32),
            pltpu.VMEM((NBUF, B, WIDTH), jnp.float32),
            pltpu.VMEM_SHARED((NACC, WIDTH), jnp.float32),
            [pltpu.SemaphoreType.DMA] * NBUF,
            [pltpu.SemaphoreType.DMA] * NBUF,
        ],
    )
    def k(xt_hbm, ei_hbm, et_hbm, out_hbm,
          srcb, typb, dstb, idxb, rowsb, acc, gsem, ssem):
        cid = lax.axis_index("c")
        sid = lax.axis_index("s")
        wid = sid * NC + cid
        row0 = wid * nchunk
        xtr = xt_hbm

        pltpu.sync_copy(ei_hbm.at[0, pl.ds(row0, nchunk)], srcb)
        pltpu.sync_copy(ei_hbm.at[1, pl.ds(row0, nchunk)], dstb)
        pltpu.sync_copy(et_hbm.at[pl.ds(row0, nchunk)], typb)

        zoffs = list(range(0, WIDTH - LANES + 1, LANES))
        if WIDTH % LANES:
            zoffs.append(WIDTH - LANES)

        @pl.loop(0, B)
        def _(i):
            for zo in zoffs:
                rowsb[0, i, pl.ds(zo, LANES)] = jnp.zeros(
                    (LANES,), jnp.float32)
        rbase = sid * rps
        for kk in range(rps // B):
            pltpu.sync_copy(rowsb.at[0], acc.at[pl.ds(rbase + kk * B, B)])

        @pl.loop(0, nchunk)
        def _(c):
            for j in range(B // LANES):
                sl = pl.ds(j * LANES, LANES)
                idxb[c, sl] = srcb[c, sl] * NTYPES + typb[c, sl]
        plsc.subcore_barrier()

        def fire_gather(c, b):
            pltpu.async_copy(xtr.at[idxb.at[c]], rowsb.at[b], gsem[b])

        def wait_gather(b):
            pltpu.make_async_copy(
                xtr.at[idxb.at[0]], rowsb.at[b], gsem[b]).wait()

        def fire_scatter(c, b):
            pltpu.async_copy(rowsb.at[b], acc.at[dstb.at[c]], ssem[b],
                             add=True)

        def wait_scatter(b):
            pltpu.make_async_copy(
                rowsb.at[b], acc.at[dstb.at[0]], ssem[b]).wait()

        for b in range(NBUF):
            fire_gather(b, b)

        @pl.loop(0, nouter - 1)
        def _(i):
            c0 = i * NBUF
            for b in range(NBUF):
                wait_gather(b)
                fire_scatter(c0 + b, b)
            for b in range(NBUF):
                wait_scatter(b)
                fire_gather(c0 + NBUF + b, b)

        c0 = (nouter - 1) * NBUF
        for b in range(NBUF):
            wait_gather(b)
            fire_scatter(c0 + b, b)
        for b in range(NBUF):
            wait_scatter(b)

        plsc.subcore_barrier()
        for kk in range(rps // B):
            r0 = rbase + kk * B
            pltpu.sync_copy(acc.at[pl.ds(r0, B)], rowsb.at[0])
            pltpu.sync_copy(rowsb.at[0],
                            out_hbm.at[cid, pl.ds(r0, B), pl.ds(0, WIDTH)])

    return k(xt2d, ei3, et2)


def _prep_conv_weights(W_conv, C, NT, WPAD):
    Cout = W_conv.shape[2]
    wx = jnp.transpose(W_conv[:, :C, :], (1, 0, 2))
    wx = jnp.pad(wx, ((0, 0), (0, 0), (0, WPAD - Cout)))
    wx = wx.reshape(C, NTYPES * WPAD)
    wo = jnp.transpose(W_conv[:, C:, :], (1, 0, 2))
    wo = jnp.pad(wo, ((0, 0), (0, 0), (0, WPAD - Cout)))
    wo = wo.reshape(NT, NTYPES * WPAD)
    bcat = jnp.zeros((NTYPES, WPAD), jnp.float32).at[:, Cout].set(1.0)
    bcat = bcat.reshape(1, NTYPES * WPAD)
    return wx, wo, bcat


def kernel(data, edge_index_0, edge_type_0, node_type_0,
           edge_index_1, edge_type_1, node_type_1, depth,
           W_down0, b_down0, gns_down0, gnb_down0,
           W_conv0, gns_conv0, gnb_conv0,
           W_down1, b_down1, gns_down1, gnb_down1,
           W_conv1, gns_conv1, gnb_conv1):
    del depth
    N0, C0 = data.shape
    N1 = node_type_0.shape[0]
    N2 = node_type_1.shape[0]
    C1 = W_down1.shape[1]
    NT0 = W_conv0.shape[1] - C0
    NT1 = W_conv1.shape[1] - C1
    W0, W1 = 40, 72
    B = 128

    wx0, wo0, bcat0 = _prep_conv_weights(W_conv0, C0, NT0, W0)
    xt0 = _tc_stage0(node_type_0.reshape(N1, 1), data, W_down0,
                     b_down0.reshape(1, C0), gns_down0.reshape(1, C0),
                     gnb_down0.reshape(1, C0), wx0, wo0, bcat0,
                     C=C0, NT=NT0, bn=2048)
    part0 = _sc_conv(xt0.reshape(N1 * NTYPES, W0), edge_index_0.reshape(2, -1, B),
                     edge_type_0.reshape(-1, B), NACC=N1, WIDTH=W0)

    wx1, wo1, bcat1 = _prep_conv_weights(W_conv1, C1, NT1, W1)
    xt1 = _tc_mid(node_type_1.reshape(N2, 1), part0[0], part0[1],
                  gns_conv0.reshape(1, C0), gnb_conv0.reshape(1, C0),
                  W_down1, b_down1.reshape(1, C1), gns_down1.reshape(1, C1),
                  gnb_down1.reshape(1, C1), wx1, wo1, bcat1,
                  C0=C0, C1=C1, NT=NT1)
    part1 = _sc_conv(xt1.reshape(N2 * NTYPES, W1), edge_index_1.reshape(2, -1, B),
                     edge_type_1.reshape(-1, B), NACC=N2, WIDTH=W1)
    out = _tc_final(part1[0], part1[1], gns_conv1.reshape(1, C1),
                    gnb_conv1.reshape(1, C1), C=C1)
    return out


# speedup vs baseline: 1.1889x; 1.0010x over previous
import functools

import jax
import jax.numpy as jnp
from jax import lax
from jax.experimental import pallas as pl
from jax.experimental.pallas import tpu as pltpu
from jax.experimental.pallas import tpu_sc as plsc

NC, NS, LANES = 2, 16, 16
NW = NC * NS
GROUPS = 8
EPS = 1e-5
NTYPES = 7


def _gn(h, gns, gnb, C):
    g = C // GROUPS
    r = lax.broadcasted_iota(jnp.int32, (C, C), 0) // g
    c = lax.broadcasted_iota(jnp.int32, (C, C), 1) // g
    mg = (r == c).astype(jnp.float32) / g
    m = jnp.dot(h, mg, preferred_element_type=jnp.float32)
    d = h - m
    v = jnp.dot(d * d, mg, preferred_element_type=jnp.float32)
    return d * lax.rsqrt(v + EPS) * gns + gnb


def _down(xin, w_ref, b_ref, gns_ref, gnb_ref, C):
    n8, cin = xin.shape
    xp = jnp.mean(xin.reshape(n8 // 8, 8, cin), axis=1)
    h = jnp.dot(xp, w_ref[:], preferred_element_type=jnp.float32) + b_ref[:]
    return jax.nn.gelu(_gn(h, gns_ref[:], gnb_ref[:], C))


def _table(x, nt_ref, wx_ref, wo_ref, bcat_ref, NT):
    oh = (nt_ref[:] == lax.broadcasted_iota(jnp.int32, (1, NT), 1))
    oh = oh.astype(jnp.float32)
    return (jnp.dot(x, wx_ref[:], preferred_element_type=jnp.float32)
            + jnp.dot(oh, wo_ref[:], preferred_element_type=jnp.float32)
            + bcat_ref[:])


def _tc_stage0_body(nt_ref, d_ref, w_ref, b_ref, gns_ref, gnb_ref,
                    wx_ref, wo_ref, bcat_ref, out_ref, *, C, NT):
    x = _down(d_ref[:], w_ref, b_ref, gns_ref, gnb_ref, C)
    out_ref[:] = _table(x, nt_ref, wx_ref, wo_ref, bcat_ref, NT)


def _tc_stage0(nt2, data, w, b, gns, gnb, wx, wo, bcat, *, C, NT, bn):
    N8, CIN = data.shape
    N = N8 // 8
    TW = wx.shape[1]
    return pl.pallas_call(
        functools.partial(_tc_stage0_body, C=C, NT=NT),
        grid=(N // bn,),
        in_specs=[
            pl.BlockSpec((bn, 1), lambda i: (i, 0)),
            pl.BlockSpec((bn * 8, CIN), lambda i: (i, 0)),
            pl.BlockSpec((CIN, C), lambda i: (0, 0)),
            pl.BlockSpec((1, C), lambda i: (0, 0)),
            pl.BlockSpec((1, C), lambda i: (0, 0)),
            pl.BlockSpec((1, C), lambda i: (0, 0)),
            pl.BlockSpec((C, TW), lambda i: (0, 0)),
            pl.BlockSpec((NT, TW), lambda i: (0, 0)),
            pl.BlockSpec((1, TW), lambda i: (0, 0)),
        ],
        out_specs=pl.BlockSpec((bn, TW), lambda i: (i, 0)),
        out_shape=jax.ShapeDtypeStruct((N, TW), jnp.float32),
    )(nt2, data, w, b, gns, gnb, wx, wo, bcat)


def _tc_mid_body(nt_ref, p0_ref, p1_ref, gnsc_ref, gnbc_ref,
                 w_ref, b_ref, gns_ref, gnb_ref, wx_ref, wo_ref, bcat_ref,
                 out_ref, *, C0, C1, NT):
    s = p0_ref[:, :C0 + 1] + p1_ref[:, :C0 + 1]
    deg = jnp.maximum(s[:, C0:C0 + 1], 1.0)
    agg = s[:, :C0] / deg
    x1 = jax.nn.gelu(_gn(agg, gnsc_ref[:], gnbc_ref[:], C0))
    x2 = _down(x1, w_ref, b_ref, gns_ref, gnb_ref, C1)
    out_ref[:] = _table(x2, nt_ref, wx_ref, wo_ref, bcat_ref, NT)


def _tc_mid(nt2, p0, p1, gnsc, gnbc, w, b, gns, gnb, wx, wo, bcat,
            *, C0, C1, NT, bn=4096):
    N1, W = p0.shape
    N2 = N1 // 8
    TW = wx.shape[1]
    PW = W
    bo = bn // 8
    return pl.pallas_call(
        functools.partial(_tc_mid_body, C0=C0, C1=C1, NT=NT),
        grid=(N1 // bn,),
        in_specs=[
            pl.BlockSpec((bo, 1), lambda i: (i, 0)),
            pl.BlockSpec((bn, PW), lambda i: (i, 0)),
            pl.BlockSpec((bn, PW), lambda i: (i, 0)),
            pl.BlockSpec((1, C0), lambda i: (0, 0)),
            pl.BlockSpec((1, C0), lambda i: (0, 0)),
            pl.BlockSpec((C0, C1), lambda i: (0, 0)),
            pl.BlockSpec((1, C1), lambda i: (0, 0)),
            pl.BlockSpec((1, C1), lambda i: (0, 0)),
            pl.BlockSpec((1, C1), lambda i: (0, 0)),
            pl.BlockSpec((C1, TW), lambda i: (0, 0)),
            pl.BlockSpec((NT, TW), lambda i: (0, 0)),
            pl.BlockSpec((1, TW), lambda i: (0, 0)),
        ],
        out_specs=pl.BlockSpec((bo, TW), lambda i: (i, 0)),
        out_shape=jax.ShapeDtypeStruct((N2, TW), jnp.float32),
    )(nt2, p0, p1, gnsc, gnbc, w, b, gns, gnb, wx, wo, bcat)


def _tc_final_body(p0_ref, p1_ref, gns_ref, gnb_ref, out_ref, *, C):
    s = p0_ref[:, :C + 1] + p1_ref[:, :C + 1]
    deg = jnp.maximum(s[:, C:C + 1], 1.0)
    agg = s[:, :C] / deg
    out_ref[:] = jax.nn.gelu(_gn(agg, gns_ref[:], gnb_ref[:], C))


def _tc_final(p0, p1, gns, gnb, *, C):
    N, W = p0.shape
    return pl.pallas_call(
        functools.partial(_tc_final_body, C=C),
        out_shape=jax.ShapeDtypeStruct((N, C), jnp.float32),
    )(p0, p1, gns, gnb)


def _sc_conv(xt2d, ei3, et2, *, NACC, WIDTH, B=128, NBUF=8):
    NTAB, TW = xt2d.shape
    E = ei3.shape[1] * B
    ew = E // NW
    nchunk = ew // B
    NBUF = min(NBUF, nchunk)
    nouter = nchunk // NBUF
    rps = NACC // NS
    mesh = plsc.VectorSubcoreMesh(core_axis_name="c", subcore_axis_name="s",
                                  num_cores=NC, num_subcores=NS)

    OUTW = 128

    @functools.partial(
        pl.kernel,
        out_type=jax.ShapeDtypeStruct((NC, NACC, OUTW), jnp.float32),
        mesh=mesh,
        compiler_params=pltpu.CompilerParams(use_tc_tiling_on_sc=False),
        scratch_types=[
            pltpu.VMEM((nchunk, B), jnp.int32),
            pltpu.VMEM((nchunk, B), jnp.int32),
            pltpu.VMEM((nchunk, B), jnp.int32),
            pltpu.VMEM((nchunk, B), jnp.int32),
            pltpu.VMEM((NBUF, B, WIDTH), jnp.float32),
            pltpu.VMEM_SHARED((NACC, WIDTH), jnp.float32),
            [pltpu.SemaphoreType.DMA] * NBUF,
            [pltpu.SemaphoreType.DMA] * NBUF,
        ],
    )
    def k(xt_hbm, ei_hbm, et_hbm, out_hbm,
          srcb, typb, dstb, idxb, rowsb, acc, gsem, ssem):
        cid = lax.axis_index("c")
        sid = lax.axis_index("s")
        wid = sid * NC + cid
        row0 = wid * nchunk
        xtr = xt_hbm

        pltpu.sync_copy(ei_hbm.at[0, pl.ds(row0, nchunk)], srcb)
        pltpu.sync_copy(ei_hbm.at[1, pl.ds(row0, nchunk)], dstb)
        pltpu.sync_copy(et_hbm.at[pl.ds(row0, nchunk)], typb)

        zoffs = list(range(0, WIDTH - LANES + 1, LANES))
        if WIDTH % LANES:
            zoffs.append(WIDTH - LANES)

        @pl.loop(0, B)
        def _(i):
            for zo in zoffs:
                rowsb[0, i, pl.ds(zo, LANES)] = jnp.zeros(
                    (LANES,), jnp.float32)
        rbase = sid * rps
        for kk in range(rps // B):
            pltpu.sync_copy(rowsb.at[0], acc.at[pl.ds(rbase + kk * B, B)])

        @pl.loop(0, nchunk)
        def _(c):
            for j in range(B // LANES):
                sl = pl.ds(j * LANES, LANES)
                idxb[c, sl] = srcb[c, sl] * NTYPES + typb[c, sl]
        plsc.subcore_barrier()

        def fire_gather(c, b):
            pltpu.async_copy(xtr.at[idxb.at[c]], rowsb.at[b], gsem[b])

        def wait_gather(b):
            pltpu.make_async_copy(
                xtr.at[idxb.at[0]], rowsb.at[b], gsem[b]).wait()

        def fire_scatter(c, b):
            pltpu.async_copy(rowsb.at[b], acc.at[dstb.at[c]], ssem[b],
                             add=True)

        def wait_scatter(b):
            pltpu.make_async_copy(
                rowsb.at[b], acc.at[dstb.at[0]], ssem[b]).wait()

        for b in range(NBUF):
            fire_gather(b, b)

        @pl.loop(0, nouter - 1)
        def _(i):
            c0 = i * NBUF
            for b in range(NBUF):
                wait_gather(b)
                fire_scatter(c0 + b, b)
            for b in range(NBUF):
                wait_scatter(b)
                fire_gather(c0 + NBUF + b, b)

        c0 = (nouter - 1) * NBUF
        for b in range(NBUF):
            wait_gather(b)
            fire_scatter(c0 + b, b)
        for b in range(NBUF):
            wait_scatter(b)

        plsc.subcore_barrier()
        for kk in range(rps // B):
            r0 = rbase + kk * B
            pltpu.sync_copy(acc.at[pl.ds(r0, B)], rowsb.at[0])
            pltpu.sync_copy(rowsb.at[0],
                            out_hbm.at[cid, pl.ds(r0, B), pl.ds(0, WIDTH)])

    return k(xt2d, ei3, et2)


def _prep_conv_weights(W_conv, C, NT, WPAD):
    Cout = W_conv.shape[2]
    wx = jnp.transpose(W_conv[:, :C, :], (1, 0, 2))
    wx = jnp.pad(wx, ((0, 0), (0, 0), (0, WPAD - Cout)))
    wx = wx.reshape(C, NTYPES * WPAD)
    wo = jnp.transpose(W_conv[:, C:, :], (1, 0, 2))
    wo = jnp.pad(wo, ((0, 0), (0, 0), (0, WPAD - Cout)))
    wo = wo.reshape(NT, NTYPES * WPAD)
    bcat = jnp.zeros((NTYPES, WPAD), jnp.float32).at[:, Cout].set(1.0)
    bcat = bcat.reshape(1, NTYPES * WPAD)
    return wx, wo, bcat


def kernel(data, edge_index_0, edge_type_0, node_type_0,
           edge_index_1, edge_type_1, node_type_1, depth,
           W_down0, b_down0, gns_down0, gnb_down0,
           W_conv0, gns_conv0, gnb_conv0,
           W_down1, b_down1, gns_down1, gnb_down1,
           W_conv1, gns_conv1, gnb_conv1):
    del depth
    N0, C0 = data.shape
    N1 = node_type_0.shape[0]
    N2 = node_type_1.shape[0]
    C1 = W_down1.shape[1]
    NT0 = W_conv0.shape[1] - C0
    NT1 = W_conv1.shape[1] - C1
    W0, W1 = 40, 72
    B = 128

    wx0, wo0, bcat0 = _prep_conv_weights(W_conv0, C0, NT0, W0)
    xt0 = _tc_stage0(node_type_0.reshape(N1, 1), data, W_down0,
                     b_down0.reshape(1, C0), gns_down0.reshape(1, C0),
                     gnb_down0.reshape(1, C0), wx0, wo0, bcat0,
                     C=C0, NT=NT0, bn=2048)
    part0 = _sc_conv(xt0.reshape(N1 * NTYPES, W0), edge_index_0.reshape(2, -1, B),
                     edge_type_0.reshape(-1, B), NACC=N1, WIDTH=W0)

    wx1, wo1, bcat1 = _prep_conv_weights(W_conv1, C1, NT1, W1)
    xt1 = _tc_mid(node_type_1.reshape(N2, 1), part0[0], part0[1],
                  gns_conv0.reshape(1, C0), gnb_conv0.reshape(1, C0),
                  W_down1, b_down1.reshape(1, C1), gns_down1.reshape(1, C1),
                  gnb_down1.reshape(1, C1), wx1, wo1, bcat1,
                  C0=C0, C1=C1, NT=NT1)
    part1 = _sc_conv(xt1.reshape(N2 * NTYPES, W1), edge_index_1.reshape(2, -1, B),
                     edge_type_1.reshape(-1, B), NACC=N2, WIDTH=W1)
    out = _tc_final(part1[0], part1[1], gns_conv1.reshape(1, C1),
                    gnb_conv1.reshape(1, C1), C=C1)
    return out
